# baseline (speedup 1.0000x reference)
import functools

import jax
import jax.numpy as jnp
from jax import lax
from jax.experimental import pallas as pl
from jax.experimental.pallas import tpu as pltpu
from jax.experimental.pallas import tpu_sc as plsc

CUTOFF = 10.0
LOG2 = 0.6931471805599453

N_NODES = 10000
N_EDGES = 320000
HIDDEN = 128
NUM_GAUSSIANS = 64

NC = 2
NS = 16
CH = 128
NCH_G = N_EDGES // CH
FULL_K = NCH_G // NS
EXTRA = NCH_G % NS
GROUP = 2500
ACC_N = 2560
STRIPE = ACC_N // NS


def _ssp(v):
    return jax.nn.softplus(v) - LOG2


def _wmlp_body(ea_ref, el_ref, w1t_ref, b1_ref, w2t_ref, b2_ref, o_ref):
    a = jnp.dot(ea_ref[...], w1t_ref[...], preferred_element_type=jnp.float32)
    a = _ssp(a + b1_ref[...])
    w = jnp.dot(a, w2t_ref[...], preferred_element_type=jnp.float32) + b2_ref[...]
    c = jnp.where(el_ref[...] <= CUTOFF, 1.0, 0.0)
    o_ref[...] = w * c


def _edge_filter(edge_attr, edge_length, nn_w1, nn_b1, nn_w2, nn_b2):
    be = 4000
    grid = N_EDGES // be
    return pl.pallas_call(
        _wmlp_body,
        grid=(grid,),
        in_specs=[
            pl.BlockSpec((be, NUM_GAUSSIANS), lambda i: (i, 0)),
            pl.BlockSpec((be, 1), lambda i: (i, 0)),
            pl.BlockSpec((NUM_GAUSSIANS, HIDDEN), lambda i: (0, 0)),
            pl.BlockSpec((1, HIDDEN), lambda i: (0, 0)),
            pl.BlockSpec((HIDDEN, HIDDEN), lambda i: (0, 0)),
            pl.BlockSpec((1, HIDDEN), lambda i: (0, 0)),
        ],
        out_specs=pl.BlockSpec((be, HIDDEN), lambda i: (i, 0)),
        out_shape=jax.ShapeDtypeStruct((N_EDGES, HIDDEN), jnp.float32),
    )(edge_attr, edge_length.reshape(N_EDGES, 1), nn_w1.T, nn_b1.reshape(1, -1),
      nn_w2.T, nn_b2.reshape(1, -1))


def _lin1_body(x_ref, w_ref, o_ref):
    o_ref[...] = jnp.dot(x_ref[...], w_ref[...], preferred_element_type=jnp.float32)


def _lin1_full(x, lin1_w):
    bn = 2000
    return pl.pallas_call(
        _lin1_body,
        grid=(N_NODES // bn,),
        in_specs=[
            pl.BlockSpec((bn, HIDDEN), lambda i: (i, 0)),
            pl.BlockSpec((HIDDEN, HIDDEN), lambda i: (0, 0)),
        ],
        out_specs=pl.BlockSpec((bn, HIDDEN), lambda i: (i, 0)),
        out_shape=jax.ShapeDtypeStruct((N_NODES, HIDDEN), jnp.float32),
    )(x, lin1_w.T)


def _sc_body(hh_hbm, w_hbm, src_hbm, dst_hbm, out_hbm,
             src_v, dst_v, idx_v, rows_v, w_v, zero_v, acc_sh, gsem):
    cid = lax.axis_index("c")
    sid = lax.axis_index("s")

    if True:
        @pl.loop(0, CH)
        def _zrow(r):
            for j in range(HIDDEN // 16):
                zero_v[r, pl.ds(j * 16, 16)] = jnp.zeros((16,), jnp.float32)

        for p in (0, 1):
            group = cid * 2 + p
            base = group * GROUP

            zs = pl.multiple_of(sid * STRIPE, 8)
            pltpu.sync_copy(zero_v, acc_sh.at[pl.ds(zs, CH)])
            pltpu.sync_copy(zero_v.at[pl.ds(0, STRIPE - CH)],
                            acc_sh.at[pl.ds(zs + CH, STRIPE - CH)])

            plsc.subcore_barrier()

            def _process(g):
                pltpu.sync_copy(src_hbm.at[g], src_v)
                pltpu.sync_copy(dst_hbm.at[g], dst_v)
                pltpu.async_copy(hh_hbm.at[src_v.at[0]], rows_v, gsem).wait()
                off = pl.multiple_of(g * CH, 8)
                pltpu.sync_copy(w_hbm.at[pl.ds(off, CH)], w_v)

                for gi in range(CH // 16):
                    sl = pl.ds(gi * 16, 16)
                    d = dst_v[0, sl]
                    rel = d - base
                    ok = (rel >= 0) & (rel < GROUP)
                    idx_v[0, sl] = jnp.where(ok, rel, GROUP + (d & 31))

                @pl.loop(0, CH)
                def _row(r):
                    for j in range(HIDDEN // 16):
                        sl = pl.ds(j * 16, 16)
                        rows_v[r, sl] = rows_v[r, sl] * w_v[r, sl]

                pltpu.sync_copy(rows_v, acc_sh.at[idx_v.at[0]], add=True)

            @pl.loop(0, FULL_K)
            def _chunk(k):
                _process(k * NS + sid)

            @pl.when(sid < EXTRA)
            def _():
                _process(FULL_K * NS + sid)

            plsc.subcore_barrier()

            fs = pl.multiple_of(sid * STRIPE, 8)
            pltpu.sync_copy(acc_sh.at[pl.ds(fs, STRIPE)],
                            out_hbm.at[group, pl.ds(fs, STRIPE)])

            plsc.subcore_barrier()


def _sc_aggregate(hh, w, src, dst):
    mesh = plsc.VectorSubcoreMesh(core_axis_name="c", subcore_axis_name="s")
    k = functools.partial(
        pl.kernel,
        out_type=jax.ShapeDtypeStruct((4, ACC_N, HIDDEN), jnp.float32),
        mesh=mesh,
        scratch_types=[
            pltpu.VMEM((1, CH), jnp.int32),
            pltpu.VMEM((1, CH), jnp.int32),
            pltpu.VMEM((1, CH), jnp.int32),
            pltpu.VMEM((CH, HIDDEN), jnp.float32),
            pltpu.VMEM((CH, HIDDEN), jnp.float32),
            pltpu.VMEM((CH, HIDDEN), jnp.float32),
            pltpu.VMEM_SHARED((ACC_N, HIDDEN), jnp.float32),
            pltpu.SemaphoreType.DMA,
        ],
    )(_sc_body)
    return k(hh, w, src.reshape(NCH_G, 1, CH), dst.reshape(NCH_G, 1, CH))


def _tail_body(p_ref, w2t_ref, b2_ref, wt_ref, b_ref, o_ref):
    t = jnp.dot(p_ref[...], w2t_ref[...], preferred_element_type=jnp.float32)
    t = _ssp(t + b2_ref[...])
    o_ref[...] = jnp.dot(t, wt_ref[...], preferred_element_type=jnp.float32) + b_ref[...]


def _tail(agg, lin2_w, lin2_b, lin_w, lin_b):
    bn = 2000
    return pl.pallas_call(
        _tail_body,
        grid=(N_NODES // bn,),
        in_specs=[
            pl.BlockSpec((bn, HIDDEN), lambda i: (i, 0)),
            pl.BlockSpec((HIDDEN, HIDDEN), lambda i: (0, 0)),
            pl.BlockSpec((1, HIDDEN), lambda i: (0, 0)),
            pl.BlockSpec((HIDDEN, HIDDEN), lambda i: (0, 0)),
            pl.BlockSpec((1, HIDDEN), lambda i: (0, 0)),
        ],
        out_specs=pl.BlockSpec((bn, HIDDEN), lambda i: (i, 0)),
        out_shape=jax.ShapeDtypeStruct((N_NODES, HIDDEN), jnp.float32),
    )(agg, lin2_w.T, lin2_b.reshape(1, -1), lin_w.T, lin_b.reshape(1, -1))


def kernel(x, edge_index, edge_length, edge_attr,
           lin1_w, nn_w1, nn_b1, nn_w2, nn_b2, lin2_w, lin2_b, lin_w, lin_b):
    w = _edge_filter(edge_attr, edge_length, nn_w1, nn_b1, nn_w2, nn_b2)
    h = _lin1_full(x, lin1_w)
    partials = _sc_aggregate(h, w, edge_index[0], edge_index[1])
    agg = partials[:, :GROUP, :].reshape(N_NODES, HIDDEN)
    return _tail(agg, lin2_w, lin2_b, lin_w, lin_b)


# speedup vs baseline: 1.3050x; 1.3050x over previous
import functools

import jax
import jax.numpy as jnp
from jax import lax
from jax.experimental import pallas as pl
from jax.experimental.pallas import tpu as pltpu
from jax.experimental.pallas import tpu_sc as plsc

CUTOFF = 10.0
LOG2 = 0.6931471805599453

N_NODES = 10000
N_EDGES = 320000
HIDDEN = 128
NUM_GAUSSIANS = 64

NC = 2
NS = 16
CH = 128
NCH_G = N_EDGES // CH
FULL_K = NCH_G // NS
EXTRA = NCH_G % NS
GROUP = 2500
ACC_N = 2560
STRIPE = ACC_N // NS


def _ssp(v):
    return jax.nn.softplus(v) - LOG2


def _wmlp_body(ea_ref, el_ref, w1t_ref, b1_ref, w2t_ref, b2_ref, o_ref):
    a = jnp.dot(ea_ref[...], w1t_ref[...], preferred_element_type=jnp.float32)
    a = _ssp(a + b1_ref[...])
    w = jnp.dot(a, w2t_ref[...], preferred_element_type=jnp.float32) + b2_ref[...]
    c = jnp.where(el_ref[...] <= CUTOFF, 1.0, 0.0)
    o_ref[...] = w * c


def _edge_filter(edge_attr, edge_length, nn_w1, nn_b1, nn_w2, nn_b2):
    be = 4000
    grid = N_EDGES // be
    return pl.pallas_call(
        _wmlp_body,
        grid=(grid,),
        in_specs=[
            pl.BlockSpec((be, NUM_GAUSSIANS), lambda i: (i, 0)),
            pl.BlockSpec((be, 1), lambda i: (i, 0)),
            pl.BlockSpec((NUM_GAUSSIANS, HIDDEN), lambda i: (0, 0)),
            pl.BlockSpec((1, HIDDEN), lambda i: (0, 0)),
            pl.BlockSpec((HIDDEN, HIDDEN), lambda i: (0, 0)),
            pl.BlockSpec((1, HIDDEN), lambda i: (0, 0)),
        ],
        out_specs=pl.BlockSpec((be, HIDDEN), lambda i: (i, 0)),
        out_shape=jax.ShapeDtypeStruct((N_EDGES, HIDDEN), jnp.float32),
    )(edge_attr, edge_length.reshape(N_EDGES, 1), nn_w1.T, nn_b1.reshape(1, -1),
      nn_w2.T, nn_b2.reshape(1, -1))


def _lin1_body(x_ref, w_ref, o_ref):
    o_ref[...] = jnp.dot(x_ref[...], w_ref[...], preferred_element_type=jnp.float32)


def _lin1_full(x, lin1_w):
    bn = 2000
    return pl.pallas_call(
        _lin1_body,
        grid=(N_NODES // bn,),
        in_specs=[
            pl.BlockSpec((bn, HIDDEN), lambda i: (i, 0)),
            pl.BlockSpec((HIDDEN, HIDDEN), lambda i: (0, 0)),
        ],
        out_specs=pl.BlockSpec((bn, HIDDEN), lambda i: (i, 0)),
        out_shape=jax.ShapeDtypeStruct((N_NODES, HIDDEN), jnp.float32),
    )(x, lin1_w.T)


def _sc_body(hh_hbm, w_hbm, src_hbm, dst_hbm, out_hbm,
             src_all, dst_all, idx_v, rows_v, w_v, zero_v, acc_sh, gsem, wsem):
    cid = lax.axis_index("c")
    sid = lax.axis_index("s")
    start_c = sid * FULL_K + jnp.minimum(sid, EXTRA)
    pltpu.sync_copy(src_hbm.at[pl.ds(start_c, FULL_K + 1)], src_all)
    pltpu.sync_copy(dst_hbm.at[pl.ds(start_c, FULL_K + 1)], dst_all)

    if True:
        @pl.loop(0, CH)
        def _zrow(r):
            for j in range(HIDDEN // 16):
                zero_v[r, pl.ds(j * 16, 16)] = jnp.zeros((16,), jnp.float32)

        for p in (0, 1):
            group = cid * 2 + p
            base = group * GROUP

            zs = pl.multiple_of(sid * STRIPE, 8)
            pltpu.sync_copy(zero_v, acc_sh.at[pl.ds(zs, CH)])
            pltpu.sync_copy(zero_v.at[pl.ds(0, STRIPE - CH)],
                            acc_sh.at[pl.ds(zs + CH, STRIPE - CH)])

            plsc.subcore_barrier()

            def _process(k):
                g = start_c + k
                gcp = pltpu.async_copy(hh_hbm.at[src_all.at[k, 0]], rows_v,
                                       gsem)
                off = pl.multiple_of(g * CH, 8)
                wcp = pltpu.async_copy(w_hbm.at[pl.ds(off, CH)], w_v, wsem)

                for gi in range(CH // 16):
                    sl = pl.ds(gi * 16, 16)
                    d = dst_all[k, 0, sl]
                    rel = d - base
                    ok = (rel >= 0) & (rel < GROUP)
                    idx_v[0, sl] = jnp.where(ok, rel, GROUP + (d & 31))

                gcp.wait()
                wcp.wait()

                @pl.loop(0, CH)
                def _row(r):
                    for j in range(HIDDEN // 16):
                        sl = pl.ds(j * 16, 16)
                        rows_v[r, sl] = rows_v[r, sl] * w_v[r, sl]

                pltpu.sync_copy(rows_v, acc_sh.at[idx_v.at[0]], add=True)

            @pl.loop(0, FULL_K)
            def _chunk(k):
                _process(k)

            @pl.when(sid < EXTRA)
            def _():
                _process(FULL_K)

            plsc.subcore_barrier()

            fs = pl.multiple_of(sid * STRIPE, 8)
            pltpu.sync_copy(acc_sh.at[pl.ds(fs, STRIPE)],
                            out_hbm.at[group, pl.ds(fs, STRIPE)])

            plsc.subcore_barrier()


def _sc_aggregate(hh, w, src, dst):
    mesh = plsc.VectorSubcoreMesh(core_axis_name="c", subcore_axis_name="s")
    k = functools.partial(
        pl.kernel,
        out_type=jax.ShapeDtypeStruct((4, ACC_N, HIDDEN), jnp.float32),
        mesh=mesh,
        scratch_types=[
            pltpu.VMEM((FULL_K + 1, 1, CH), jnp.int32),
            pltpu.VMEM((FULL_K + 1, 1, CH), jnp.int32),
            pltpu.VMEM((1, CH), jnp.int32),
            pltpu.VMEM((CH, HIDDEN), jnp.float32),
            pltpu.VMEM((CH, HIDDEN), jnp.float32),
            pltpu.VMEM((CH, HIDDEN), jnp.float32),
            pltpu.VMEM_SHARED((ACC_N, HIDDEN), jnp.float32),
            pltpu.SemaphoreType.DMA,
            pltpu.SemaphoreType.DMA,
        ],
    )(_sc_body)
    pad = jnp.zeros((1, 1, CH), jnp.int32)
    src3 = jnp.concatenate([src.reshape(NCH_G, 1, CH), pad])
    dst3 = jnp.concatenate([dst.reshape(NCH_G, 1, CH), pad])
    return k(hh, w, src3, dst3)


def _tail_body(p_ref, w2t_ref, b2_ref, wt_ref, b_ref, o_ref):
    t = jnp.dot(p_ref[...], w2t_ref[...], preferred_element_type=jnp.float32)
    t = _ssp(t + b2_ref[...])
    o_ref[...] = jnp.dot(t, wt_ref[...], preferred_element_type=jnp.float32) + b_ref[...]


def _tail(agg, lin2_w, lin2_b, lin_w, lin_b):
    bn = 2000
    return pl.pallas_call(
        _tail_body,
        grid=(N_NODES // bn,),
        in_specs=[
            pl.BlockSpec((bn, HIDDEN), lambda i: (i, 0)),
            pl.BlockSpec((HIDDEN, HIDDEN), lambda i: (0, 0)),
            pl.BlockSpec((1, HIDDEN), lambda i: (0, 0)),
            pl.BlockSpec((HIDDEN, HIDDEN), lambda i: (0, 0)),
            pl.BlockSpec((1, HIDDEN), lambda i: (0, 0)),
        ],
        out_specs=pl.BlockSpec((bn, HIDDEN), lambda i: (i, 0)),
        out_shape=jax.ShapeDtypeStruct((N_NODES, HIDDEN), jnp.float32),
    )(agg, lin2_w.T, lin2_b.reshape(1, -1), lin_w.T, lin_b.reshape(1, -1))


def kernel(x, edge_index, edge_length, edge_attr,
           lin1_w, nn_w1, nn_b1, nn_w2, nn_b2, lin2_w, lin2_b, lin_w, lin_b):
    w = _edge_filter(edge_attr, edge_length, nn_w1, nn_b1, nn_w2, nn_b2)
    h = _lin1_full(x, lin1_w)
    partials = _sc_aggregate(h, w, edge_index[0], edge_index[1])
    agg = partials[:, :GROUP, :].reshape(N_NODES, HIDDEN)
    return _tail(agg, lin2_w, lin2_b, lin_w, lin_b)


# speedup vs baseline: 1.8678x; 1.4313x over previous
import functools

import jax
import jax.numpy as jnp
from jax import lax
from jax.experimental import pallas as pl
from jax.experimental.pallas import tpu as pltpu
from jax.experimental.pallas import tpu_sc as plsc

CUTOFF = 10.0
LOG2 = 0.6931471805599453

N_NODES = 10000
N_EDGES = 320000
HIDDEN = 128
NUM_GAUSSIANS = 64

NC = 2
NS = 16
CH = 128
NCH_G = N_EDGES // CH
FULL_K = NCH_G // NS
EXTRA = NCH_G % NS
GROUP = 2500
ACC_N = 2560
STRIPE = ACC_N // NS


def _ssp(v):
    return jax.nn.softplus(v) - LOG2


def _wmlp_body(ea_ref, el_ref, w1t_ref, b1_ref, w2t_ref, b2_ref, o_ref):
    a = jnp.dot(ea_ref[...], w1t_ref[...], preferred_element_type=jnp.float32)
    a = _ssp(a + b1_ref[...])
    w = jnp.dot(a, w2t_ref[...], preferred_element_type=jnp.float32) + b2_ref[...]
    c = jnp.where(el_ref[...] <= CUTOFF, 1.0, 0.0)
    o_ref[...] = w * c


def _edge_filter(edge_attr, edge_length, nn_w1, nn_b1, nn_w2, nn_b2):
    be = 4000
    grid = N_EDGES // be
    return pl.pallas_call(
        _wmlp_body,
        grid=(grid,),
        in_specs=[
            pl.BlockSpec((be, NUM_GAUSSIANS), lambda i: (i, 0)),
            pl.BlockSpec((be, 1), lambda i: (i, 0)),
            pl.BlockSpec((NUM_GAUSSIANS, HIDDEN), lambda i: (0, 0)),
            pl.BlockSpec((1, HIDDEN), lambda i: (0, 0)),
            pl.BlockSpec((HIDDEN, HIDDEN), lambda i: (0, 0)),
            pl.BlockSpec((1, HIDDEN), lambda i: (0, 0)),
        ],
        out_specs=pl.BlockSpec((be, HIDDEN), lambda i: (i, 0)),
        out_shape=jax.ShapeDtypeStruct((N_EDGES, HIDDEN), jnp.float32),
    )(edge_attr, edge_length.reshape(N_EDGES, 1), nn_w1.T, nn_b1.reshape(1, -1),
      nn_w2.T, nn_b2.reshape(1, -1))


def _lin1_body(x_ref, w_ref, o_ref):
    o_ref[...] = jnp.dot(x_ref[...], w_ref[...], preferred_element_type=jnp.float32)


def _lin1_full(x, lin1_w):
    bn = 2000
    return pl.pallas_call(
        _lin1_body,
        grid=(N_NODES // bn,),
        in_specs=[
            pl.BlockSpec((bn, HIDDEN), lambda i: (i, 0)),
            pl.BlockSpec((HIDDEN, HIDDEN), lambda i: (0, 0)),
        ],
        out_specs=pl.BlockSpec((bn, HIDDEN), lambda i: (i, 0)),
        out_shape=jax.ShapeDtypeStruct((N_NODES, HIDDEN), jnp.float32),
    )(x, lin1_w.T)


def _sc_body(hh_hbm, w_hbm, src_hbm, dst_hbm, out_hbm,
             src_all, dst_all, idx_v, rows_a, w_a, rows_b, w_b, zero_v,
             acc_sh, gsa, wsa, gsb, wsb):
    cid = lax.axis_index("c")
    sid = lax.axis_index("s")
    start_c = sid * FULL_K + jnp.minimum(sid, EXTRA)
    pltpu.sync_copy(src_hbm.at[pl.ds(start_c, FULL_K + 1)], src_all)
    pltpu.sync_copy(dst_hbm.at[pl.ds(start_c, FULL_K + 1)], dst_all)

    if True:
        @pl.loop(0, 32)
        def _zrow(r):
            for j in range(HIDDEN // 16):
                zero_v[r, pl.ds(j * 16, 16)] = jnp.zeros((16,), jnp.float32)

        for p in (0, 1):
            group = cid * 2 + p
            base = group * GROUP

            zs = pl.multiple_of(sid * STRIPE, 8)
            for zi in range(STRIPE // 32):
                pltpu.sync_copy(zero_v, acc_sh.at[pl.ds(zs + zi * 32, 32)])

            plsc.subcore_barrier()

            def _issue(k, rows_v, w_v, gsem, wsem):
                g = jnp.minimum(start_c + k, NCH_G - 1)
                kk = jnp.minimum(k, FULL_K)
                pltpu.async_copy(hh_hbm.at[src_all.at[kk, 0]], rows_v, gsem)
                off = pl.multiple_of(g * CH, 8)
                pltpu.async_copy(w_hbm.at[pl.ds(off, CH)], w_v, wsem)

            def _wait(rows_v, w_v, gsem, wsem):
                pltpu.make_async_copy(hh_hbm.at[src_all.at[0, 0]], rows_v,
                                      gsem).wait()
                pltpu.make_async_copy(w_hbm.at[pl.ds(0, CH)], w_v, wsem).wait()

            def _finish(k, rows_v, w_v, gsem, wsem):
                for gi in range(CH // 16):
                    sl = pl.ds(gi * 16, 16)
                    d = dst_all[k, 0, sl]
                    rel = d - base
                    ok = (rel >= 0) & (rel < GROUP)
                    idx_v[0, sl] = jnp.where(ok, rel, GROUP + (d & 31))

                _wait(rows_v, w_v, gsem, wsem)

                @pl.loop(0, CH)
                def _row(r):
                    for j in range(HIDDEN // 16):
                        sl = pl.ds(j * 16, 16)
                        rows_v[r, sl] = rows_v[r, sl] * w_v[r, sl]

                pltpu.sync_copy(rows_v, acc_sh.at[idx_v.at[0]], add=True)

            _issue(0, rows_a, w_a, gsa, wsa)

            @pl.loop(0, FULL_K // 2)
            def _pair(i):
                k = i * 2
                _issue(k + 1, rows_b, w_b, gsb, wsb)
                _finish(k, rows_a, w_a, gsa, wsa)
                _issue(k + 2, rows_a, w_a, gsa, wsa)
                _finish(k + 1, rows_b, w_b, gsb, wsb)

            @pl.when(sid < EXTRA)
            def _():
                _finish(FULL_K, rows_a, w_a, gsa, wsa)

            @pl.when(sid >= EXTRA)
            def _():
                _wait(rows_a, w_a, gsa, wsa)

            plsc.subcore_barrier()

            fs = pl.multiple_of(sid * STRIPE, 8)
            pltpu.sync_copy(acc_sh.at[pl.ds(fs, STRIPE)],
                            out_hbm.at[group, pl.ds(fs, STRIPE)])

            plsc.subcore_barrier()


def _sc_aggregate(hh, w, src, dst):
    mesh = plsc.VectorSubcoreMesh(core_axis_name="c", subcore_axis_name="s")
    k = functools.partial(
        pl.kernel,
        out_type=jax.ShapeDtypeStruct((4, ACC_N, HIDDEN), jnp.float32),
        mesh=mesh,
        scratch_types=[
            pltpu.VMEM((FULL_K + 1, 1, CH), jnp.int32),
            pltpu.VMEM((FULL_K + 1, 1, CH), jnp.int32),
            pltpu.VMEM((1, CH), jnp.int32),
            pltpu.VMEM((CH, HIDDEN), jnp.float32),
            pltpu.VMEM((CH, HIDDEN), jnp.float32),
            pltpu.VMEM((CH, HIDDEN), jnp.float32),
            pltpu.VMEM((CH, HIDDEN), jnp.float32),
            pltpu.VMEM((32, HIDDEN), jnp.float32),
            pltpu.VMEM_SHARED((ACC_N, HIDDEN), jnp.float32),
            pltpu.SemaphoreType.DMA,
            pltpu.SemaphoreType.DMA,
            pltpu.SemaphoreType.DMA,
            pltpu.SemaphoreType.DMA,
        ],
    )(_sc_body)
    pad = jnp.zeros((1, 1, CH), jnp.int32)
    src3 = jnp.concatenate([src.reshape(NCH_G, 1, CH), pad])
    dst3 = jnp.concatenate([dst.reshape(NCH_G, 1, CH), pad])
    return k(hh, w, src3, dst3)


def _tail_body(p_ref, w2t_ref, b2_ref, wt_ref, b_ref, o_ref):
    t = jnp.dot(p_ref[...], w2t_ref[...], preferred_element_type=jnp.float32)
    t = _ssp(t + b2_ref[...])
    o_ref[...] = jnp.dot(t, wt_ref[...], preferred_element_type=jnp.float32) + b_ref[...]


def _tail(agg, lin2_w, lin2_b, lin_w, lin_b):
    bn = 2000
    return pl.pallas_call(
        _tail_body,
        grid=(N_NODES // bn,),
        in_specs=[
            pl.BlockSpec((bn, HIDDEN), lambda i: (i, 0)),
            pl.BlockSpec((HIDDEN, HIDDEN), lambda i: (0, 0)),
            pl.BlockSpec((1, HIDDEN), lambda i: (0, 0)),
            pl.BlockSpec((HIDDEN, HIDDEN), lambda i: (0, 0)),
            pl.BlockSpec((1, HIDDEN), lambda i: (0, 0)),
        ],
        out_specs=pl.BlockSpec((bn, HIDDEN), lambda i: (i, 0)),
        out_shape=jax.ShapeDtypeStruct((N_NODES, HIDDEN), jnp.float32),
    )(agg, lin2_w.T, lin2_b.reshape(1, -1), lin_w.T, lin_b.reshape(1, -1))


def kernel(x, edge_index, edge_length, edge_attr,
           lin1_w, nn_w1, nn_b1, nn_w2, nn_b2, lin2_w, lin2_b, lin_w, lin_b):
    w = _edge_filter(edge_attr, edge_length, nn_w1, nn_b1, nn_w2, nn_b2)
    h = _lin1_full(x, lin1_w)
    partials = _sc_aggregate(h, w, edge_index[0], edge_index[1])
    agg = partials[:, :GROUP, :].reshape(N_NODES, HIDDEN)
    return _tail(agg, lin2_w, lin2_b, lin_w, lin_b)
